# Initial kernel scaffold; baseline (speedup 1.0000x reference)
import functools

import jax
import jax.numpy as jnp
from jax import lax
from jax.experimental import pallas as pl
from jax.experimental.pallas import tpu as pltpu
from jax.experimental.pallas import tpu_sc as plsc

N_NODES = 10000
N_EDGES = 320000
D = 128
D_EDGE = 16

NC = 2
NS = 16
NW = NC * NS
E_PER_W = N_EDGES // NW
CHUNK = 80
N_CHUNKS = E_PER_W // CHUNK
ROWS_PER_TILE = N_NODES // NS
ZCH = 125


def _proj_body(n_ref, ws_ref, wr_ref, ps_ref, pr_ref):
    x = n_ref[...]
    ps_ref[...] = jnp.dot(x, ws_ref[...], preferred_element_type=jnp.float32)
    pr_ref[...] = jnp.dot(x, wr_ref[...], preferred_element_type=jnp.float32)


def _ebase_body(e_ref, w_ref, b_ref, o_ref):
    o_ref[...] = (
        jnp.dot(e_ref[...], w_ref[...], preferred_element_type=jnp.float32)
        + b_ref[...]
    )


def _node_body(n_ref, a_ref, w1_ref, w2_ref, b_ref, o_ref):
    agg = a_ref[0] + a_ref[1]
    o_ref[...] = jnp.maximum(
        jnp.dot(n_ref[...], w1_ref[...], preferred_element_type=jnp.float32)
        + jnp.dot(agg, w2_ref[...], preferred_element_type=jnp.float32)
        + b_ref[...],
        0.0,
    )


def _sc_body(ps_hbm, pr_hbm, eb_hbm, s_hbm, r_hbm,
             ne_hbm, agg_hbm,
             sidx_v, ridx_v, ps_v, pr_v, e_v, out_v, z_v, agg_sh,
             sem_s, sem_r, sem_e):
    c = lax.axis_index("c")
    s = lax.axis_index("s")
    wid = s * NC + c
    edge0 = wid * E_PER_W

    def zrow(i, _):
        for j in range(D // 16):
            z_v[i, pl.ds(j * 16, 16)] = jnp.zeros((16,), jnp.float32)
        return 0
    lax.fori_loop(0, ZCH, zrow, 0)
    for k in range(ROWS_PER_TILE // ZCH):
        pltpu.sync_copy(z_v, agg_sh.at[pl.ds(s * ROWS_PER_TILE + k * ZCH, ZCH)])
    plsc.subcore_barrier()

    def chunk_body(k, _):
        base = edge0 + k * CHUNK
        pltpu.sync_copy(s_hbm.at[pl.ds(base, CHUNK)], sidx_v)
        pltpu.sync_copy(r_hbm.at[pl.ds(base, CHUNK)], ridx_v)
        cp_s = pltpu.async_copy(ps_hbm.at[sidx_v], ps_v, sem_s)
        cp_r = pltpu.async_copy(pr_hbm.at[ridx_v], pr_v, sem_r)
        cp_e = pltpu.async_copy(eb_hbm.at[pl.ds(base, CHUNK)], e_v, sem_e)
        cp_s.wait()
        cp_r.wait()
        cp_e.wait()

        def row(i, _):
            for j in range(D // 16):
                sl = pl.ds(j * 16, 16)
                out_v[i, sl] = jnp.maximum(
                    e_v[i, sl] + ps_v[i, sl] + pr_v[i, sl], 0.0)
            return 0
        lax.fori_loop(0, CHUNK, row, 0)

        pltpu.sync_copy(out_v, ne_hbm.at[pl.ds(base, CHUNK)])
        pltpu.sync_copy(out_v, agg_sh.at[ridx_v], add=True)
        return 0
    lax.fori_loop(0, N_CHUNKS, chunk_body, 0)

    plsc.subcore_barrier()
    for k in range(ROWS_PER_TILE // ZCH):
        off = s * ROWS_PER_TILE + k * ZCH
        pltpu.sync_copy(agg_sh.at[pl.ds(off, ZCH)],
                        agg_hbm.at[c, pl.ds(off, ZCH)])


_sc_edges = pl.kernel(
    _sc_body,
    out_type=[
        jax.ShapeDtypeStruct((N_EDGES, D), jnp.float32),
        jax.ShapeDtypeStruct((NC, N_NODES, D), jnp.float32),
    ],
    mesh=plsc.VectorSubcoreMesh(core_axis_name="c", subcore_axis_name="s"),
    scratch_types=[
        pltpu.VMEM((CHUNK,), jnp.int32),
        pltpu.VMEM((CHUNK,), jnp.int32),
        pltpu.VMEM((CHUNK, D), jnp.float32),
        pltpu.VMEM((CHUNK, D), jnp.float32),
        pltpu.VMEM((CHUNK, D), jnp.float32),
        pltpu.VMEM((CHUNK, D), jnp.float32),
        pltpu.VMEM((ZCH, D), jnp.float32),
        pltpu.VMEM_SHARED((N_NODES, D), jnp.float32),
        pltpu.SemaphoreType.DMA,
        pltpu.SemaphoreType.DMA,
        pltpu.SemaphoreType.DMA,
    ],
)


@jax.jit
def _run(nodes, edges, senders, receivers, W_edge, b_edge, W_node, b_node):
    w1 = W_edge[:D_EDGE]
    w_es = W_edge[D_EDGE:D_EDGE + D]
    w_er = W_edge[D_EDGE + D:]
    wn1 = W_node[:D]
    wn2 = W_node[D:]
    be = b_edge.reshape(1, D)
    bn = b_node.reshape(1, D)

    nb = 1000
    ps, pr = pl.pallas_call(
        _proj_body,
        grid=(N_NODES // nb,),
        in_specs=[
            pl.BlockSpec((nb, D), lambda i: (i, 0)),
            pl.BlockSpec((D, D), lambda i: (0, 0)),
            pl.BlockSpec((D, D), lambda i: (0, 0)),
        ],
        out_specs=[
            pl.BlockSpec((nb, D), lambda i: (i, 0)),
            pl.BlockSpec((nb, D), lambda i: (i, 0)),
        ],
        out_shape=[
            jax.ShapeDtypeStruct((N_NODES, D), jnp.float32),
            jax.ShapeDtypeStruct((N_NODES, D), jnp.float32),
        ],
    )(nodes, w_es, w_er)

    eb = 3200
    e_base = pl.pallas_call(
        _ebase_body,
        grid=(N_EDGES // eb,),
        in_specs=[
            pl.BlockSpec((eb, D_EDGE), lambda i: (i, 0)),
            pl.BlockSpec((D_EDGE, D), lambda i: (0, 0)),
            pl.BlockSpec((1, D), lambda i: (0, 0)),
        ],
        out_specs=pl.BlockSpec((eb, D), lambda i: (i, 0)),
        out_shape=jax.ShapeDtypeStruct((N_EDGES, D), jnp.float32),
    )(edges, w1, be)

    new_edges, agg2 = _sc_edges(ps, pr, e_base, senders, receivers)

    new_nodes = pl.pallas_call(
        _node_body,
        grid=(N_NODES // nb,),
        in_specs=[
            pl.BlockSpec((nb, D), lambda i: (i, 0)),
            pl.BlockSpec((NC, nb, D), lambda i: (0, i, 0)),
            pl.BlockSpec((D, D), lambda i: (0, 0)),
            pl.BlockSpec((D, D), lambda i: (0, 0)),
            pl.BlockSpec((1, D), lambda i: (0, 0)),
        ],
        out_specs=pl.BlockSpec((nb, D), lambda i: (i, 0)),
        out_shape=jax.ShapeDtypeStruct((N_NODES, D), jnp.float32),
    )(nodes, agg2, wn1, wn2, bn)

    return new_nodes, new_edges


def kernel(nodes, edges, senders, receivers, W_edge, b_edge, W_node, b_node):
    return _run(nodes, edges, senders, receivers,
                W_edge, b_edge, W_node, b_node)


# speedup vs baseline: 2.4157x; 2.4157x over previous
import functools

import jax
import jax.numpy as jnp
from jax import lax
from jax.experimental import pallas as pl
from jax.experimental.pallas import tpu as pltpu
from jax.experimental.pallas import tpu_sc as plsc

N_NODES = 10000
N_EDGES = 320000
D = 128
D_EDGE = 16

NC = 2
NS = 16
NW = NC * NS
E_PER_W = N_EDGES // NW
CHUNK = 80
N_CHUNKS = E_PER_W // CHUNK
ZCH = 80
N_ZCH = N_NODES // ZCH
ZCH_PER_TILE = -(-N_ZCH // NS)


def _proj_body(n_ref, ws_ref, wr_ref, ps_ref, pr_ref):
    x = n_ref[...]
    ps_ref[...] = jnp.dot(x, ws_ref[...], preferred_element_type=jnp.float32)
    pr_ref[...] = jnp.dot(x, wr_ref[...], preferred_element_type=jnp.float32)


def _ebase_body(e_ref, w_ref, b_ref, o_ref):
    o_ref[...] = (
        jnp.dot(e_ref[...], w_ref[...], preferred_element_type=jnp.float32)
        + b_ref[...]
    )


def _node_body(n_ref, a_ref, w1_ref, w2_ref, b_ref, o_ref):
    agg = a_ref[0] + a_ref[1]
    o_ref[...] = jnp.maximum(
        jnp.dot(n_ref[...], w1_ref[...], preferred_element_type=jnp.float32)
        + jnp.dot(agg, w2_ref[...], preferred_element_type=jnp.float32)
        + b_ref[...],
        0.0,
    )


def _sc_body(ps_hbm, pr_hbm, eb_hbm, s_hbm, r_hbm,
             ne_hbm, agg_hbm,
             sidx_v, ridx_v, ps_v, pr_v, e_v, out_v, agg_sh,
             sem_s, sem_r, sem_e):
    c = lax.axis_index("c")
    s = lax.axis_index("s")
    wid = s * NC + c
    edge0 = wid * E_PER_W

    def zrow(i, _):
        for j in range(D // 16):
            out_v[i, pl.ds(j * 16, 16)] = jnp.zeros((16,), jnp.float32)
        return 0
    lax.fori_loop(0, ZCH, zrow, 0)
    for k in range(ZCH_PER_TILE):
        cid = s + NS * k
        @pl.when(cid < N_ZCH)
        def _():
            pltpu.sync_copy(out_v, agg_sh.at[pl.ds(cid * ZCH, ZCH)])
    plsc.subcore_barrier()

    def chunk_body(k, _):
        base = edge0 + k * CHUNK
        pltpu.sync_copy(s_hbm.at[pl.ds(base, CHUNK)], sidx_v)
        pltpu.sync_copy(r_hbm.at[pl.ds(base, CHUNK)], ridx_v)
        cp_s = pltpu.async_copy(ps_hbm.at[sidx_v], ps_v, sem_s)
        cp_r = pltpu.async_copy(pr_hbm.at[ridx_v], pr_v, sem_r)
        cp_e = pltpu.async_copy(eb_hbm.at[pl.ds(base, CHUNK)], e_v, sem_e)
        cp_s.wait()
        cp_r.wait()
        cp_e.wait()

        def row(i, _):
            for j in range(D // 16):
                sl = pl.ds(j * 16, 16)
                out_v[i, sl] = jnp.maximum(
                    e_v[i, sl] + ps_v[i, sl] + pr_v[i, sl], 0.0)
            return 0
        lax.fori_loop(0, CHUNK, row, 0)

        pltpu.sync_copy(out_v, ne_hbm.at[pl.ds(base, CHUNK)])
        pltpu.sync_copy(out_v, agg_sh.at[ridx_v], add=True)
        return 0
    lax.fori_loop(0, N_CHUNKS, chunk_body, 0)

    plsc.subcore_barrier()
    for k in range(ZCH_PER_TILE):
        cid = s + NS * k
        @pl.when(cid < N_ZCH)
        def _():
            pltpu.sync_copy(agg_sh.at[pl.ds(cid * ZCH, ZCH)],
                            agg_hbm.at[c, pl.ds(cid * ZCH, ZCH)])


_sc_edges = pl.kernel(
    _sc_body,
    out_type=[
        jax.ShapeDtypeStruct((N_EDGES, D), jnp.float32),
        jax.ShapeDtypeStruct((NC, N_NODES, D), jnp.float32),
    ],
    mesh=plsc.VectorSubcoreMesh(core_axis_name="c", subcore_axis_name="s"),
    scratch_types=[
        pltpu.VMEM((CHUNK,), jnp.int32),
        pltpu.VMEM((CHUNK,), jnp.int32),
        pltpu.VMEM((CHUNK, D), jnp.float32),
        pltpu.VMEM((CHUNK, D), jnp.float32),
        pltpu.VMEM((CHUNK, D), jnp.float32),
        pltpu.VMEM((CHUNK, D), jnp.float32),
        pltpu.VMEM_SHARED((N_NODES, D), jnp.float32),
        pltpu.SemaphoreType.DMA,
        pltpu.SemaphoreType.DMA,
        pltpu.SemaphoreType.DMA,
    ],
)


@jax.jit
def _run(nodes, edges, senders, receivers, W_edge, b_edge, W_node, b_node):
    w1 = W_edge[:D_EDGE]
    w_es = W_edge[D_EDGE:D_EDGE + D]
    w_er = W_edge[D_EDGE + D:]
    wn1 = W_node[:D]
    wn2 = W_node[D:]
    be = b_edge.reshape(1, D)
    bn = b_node.reshape(1, D)

    nb = 1000
    ps, pr = pl.pallas_call(
        _proj_body,
        grid=(N_NODES // nb,),
        in_specs=[
            pl.BlockSpec((nb, D), lambda i: (i, 0)),
            pl.BlockSpec((D, D), lambda i: (0, 0)),
            pl.BlockSpec((D, D), lambda i: (0, 0)),
        ],
        out_specs=[
            pl.BlockSpec((nb, D), lambda i: (i, 0)),
            pl.BlockSpec((nb, D), lambda i: (i, 0)),
        ],
        out_shape=[
            jax.ShapeDtypeStruct((N_NODES, D), jnp.float32),
            jax.ShapeDtypeStruct((N_NODES, D), jnp.float32),
        ],
    )(nodes, w_es, w_er)

    eb = 3200
    e_base = pl.pallas_call(
        _ebase_body,
        grid=(N_EDGES // eb,),
        in_specs=[
            pl.BlockSpec((eb, D_EDGE), lambda i: (i, 0)),
            pl.BlockSpec((D_EDGE, D), lambda i: (0, 0)),
            pl.BlockSpec((1, D), lambda i: (0, 0)),
        ],
        out_specs=pl.BlockSpec((eb, D), lambda i: (i, 0)),
        out_shape=jax.ShapeDtypeStruct((N_EDGES, D), jnp.float32),
    )(edges, w1, be)

    new_edges, agg2 = _sc_edges(ps, pr, e_base, senders, receivers)

    new_nodes = pl.pallas_call(
        _node_body,
        grid=(N_NODES // nb,),
        in_specs=[
            pl.BlockSpec((nb, D), lambda i: (i, 0)),
            pl.BlockSpec((NC, nb, D), lambda i: (0, i, 0)),
            pl.BlockSpec((D, D), lambda i: (0, 0)),
            pl.BlockSpec((D, D), lambda i: (0, 0)),
            pl.BlockSpec((1, D), lambda i: (0, 0)),
        ],
        out_specs=pl.BlockSpec((nb, D), lambda i: (i, 0)),
        out_shape=jax.ShapeDtypeStruct((N_NODES, D), jnp.float32),
    )(nodes, agg2, wn1, wn2, bn)

    return new_nodes, new_edges


def kernel(nodes, edges, senders, receivers, W_edge, b_edge, W_node, b_node):
    return _run(nodes, edges, senders, receivers,
                W_edge, b_edge, W_node, b_node)


# speedup vs baseline: 2.7923x; 1.1559x over previous
import functools

import jax
import jax.numpy as jnp
from jax import lax
from jax.experimental import pallas as pl
from jax.experimental.pallas import tpu as pltpu
from jax.experimental.pallas import tpu_sc as plsc

N_NODES = 10000
N_EDGES = 320000
D = 128
D_EDGE = 16

NC = 2
NS = 16
NW = NC * NS
E_PER_W = N_EDGES // NW
CHUNK = 80
N_CHUNKS = E_PER_W // CHUNK
N_PAIR = N_CHUNKS // 2

AGG_BE = 1000
AGG_NB = N_EDGES // AGG_BE
AGG_W = 128


def _proj_body(n_ref, ws_ref, wr_ref, ps_ref, pr_ref):
    x = n_ref[...]
    ps_ref[...] = jnp.dot(x, ws_ref[...], preferred_element_type=jnp.float32)
    pr_ref[...] = jnp.dot(x, wr_ref[...], preferred_element_type=jnp.float32)


def _ebase_body(e_ref, w_ref, b_ref, o_ref):
    o_ref[...] = (
        jnp.dot(e_ref[...], w_ref[...], preferred_element_type=jnp.float32)
        + b_ref[...]
    )


def _node_body(n_ref, a_ref, w1_ref, w2_ref, b_ref, o_ref):
    o_ref[...] = jnp.maximum(
        jnp.dot(n_ref[...], w1_ref[...], preferred_element_type=jnp.float32)
        + jnp.dot(a_ref[...], w2_ref[...], preferred_element_type=jnp.float32)
        + b_ref[...],
        0.0,
    )


def _sc_body(ps_hbm, pr_hbm, eb_hbm, s_hbm, r_hbm,
             ne_hbm,
             sidx_f, ridx_f,
             ps0, pr0, e0, ps1, pr1, e1, out0, out1,
             sem_s0, sem_r0, sem_e0, sem_s1, sem_r1, sem_e1,
             sem_ne0, sem_ne1):
    c = lax.axis_index("c")
    s = lax.axis_index("s")
    wid = s * NC + c
    edge0 = wid * E_PER_W
    set0, sems0 = (ps0, pr0, e0), (sem_s0, sem_r0, sem_e0)
    set1, sems1 = (ps1, pr1, e1), (sem_s1, sem_r1, sem_e1)

    def mk_in(k, bufs, sems):
        ps_v, pr_v, e_v = bufs
        ss, sr, se = sems
        base = edge0 + k * CHUNK
        return (
            pltpu.make_async_copy(
                ps_hbm.at[sidx_f.at[pl.ds(k * CHUNK, CHUNK)]], ps_v, ss),
            pltpu.make_async_copy(
                pr_hbm.at[ridx_f.at[pl.ds(k * CHUNK, CHUNK)]], pr_v, sr),
            pltpu.make_async_copy(eb_hbm.at[pl.ds(base, CHUNK)], e_v, se),
        )

    def start_in(k, bufs, sems):
        for cp in mk_in(k, bufs, sems):
            cp.start()

    def wait_in(k, bufs, sems):
        for cp in mk_in(k, bufs, sems):
            cp.wait()

    def compute(bufs, out_v):
        ps_v, pr_v, e_v = bufs
        def row(i, _):
            for j in range(D // 16):
                sl = pl.ds(j * 16, 16)
                out_v[i, sl] = jnp.maximum(
                    e_v[i, sl] + ps_v[i, sl] + pr_v[i, sl], 0.0)
            return 0
        lax.fori_loop(0, CHUNK, row, 0)

    def start_ne(k, out_v, sem_ne):
        base = edge0 + k * CHUNK
        pltpu.async_copy(out_v, ne_hbm.at[pl.ds(base, CHUNK)], sem_ne)

    def wait_ne(k, out_v, sem_ne):
        base = edge0 + k * CHUNK
        pltpu.make_async_copy(out_v, ne_hbm.at[pl.ds(base, CHUNK)],
                              sem_ne).wait()

    pltpu.sync_copy(s_hbm.at[pl.ds(edge0, E_PER_W)], sidx_f)
    pltpu.sync_copy(r_hbm.at[pl.ds(edge0, E_PER_W)], ridx_f)

    start_in(0, set0, sems0)

    def pair(p, _):
        k0 = 2 * p
        k1 = 2 * p + 1
        start_in(k1, set1, sems1)

        @pl.when(p > 0)
        def _():
            wait_ne(k0, out0, sem_ne0)
        wait_in(k0, set0, sems0)
        compute(set0, out0)
        start_ne(k0, out0, sem_ne0)
        start_in(k0 + 2, set0, sems0)

        @pl.when(p > 0)
        def _():
            wait_ne(k1, out1, sem_ne1)
        wait_in(k1, set1, sems1)
        compute(set1, out1)
        start_ne(k1, out1, sem_ne1)
        return 0
    lax.fori_loop(0, N_PAIR, pair, 0)

    kt = N_CHUNKS - 1
    wait_ne(kt, out0, sem_ne0)
    wait_ne(kt, out1, sem_ne1)
    wait_in(kt, set0, sems0)
    compute(set0, out0)
    start_ne(kt, out0, sem_ne0)
    wait_ne(kt, out0, sem_ne0)


def _agg_body(r_ref, ne_ref, agg_ref):
    i = pl.program_id(0)

    @pl.when(i == 0)
    def _():
        agg_ref[...] = jnp.zeros_like(agg_ref)

    r = r_ref[0, 0, :]
    ne = ne_ref[...]
    hi = ne.astype(jnp.bfloat16)
    lo = (ne - hi.astype(jnp.float32)).astype(jnp.bfloat16)
    hilo = jnp.concatenate([hi, lo], axis=1)

    r0 = r[0]
    rmax = r[AGG_BE - 1]
    w0 = jnp.minimum((r0 // 8) * 8, N_NODES - AGG_W)
    nwin = (rmax - w0) // AGG_W + 1

    def win(k, _):
        lob = w0 + k * AGG_W
        wk = jnp.minimum(lob, N_NODES - AGG_W)
        cols = jax.lax.broadcasted_iota(jnp.int32, (AGG_BE, AGG_W), 1) + wk
        rr = r[:, None]
        oh = ((rr == cols) & (rr >= lob) & (rr < lob + AGG_W))
        part = jax.lax.dot_general(
            oh.astype(jnp.bfloat16), hilo, (((0,), (0,)), ((), ())),
            preferred_element_type=jnp.float32)
        agg_ref[pl.ds(wk, AGG_W), :] += part[:, :D] + part[:, D:]
        return 0
    lax.fori_loop(0, nwin, win, 0)


@functools.lru_cache(maxsize=None)
def _get_sc_edges():
  return pl.kernel(
    _sc_body,
    out_type=jax.ShapeDtypeStruct((N_EDGES, D), jnp.float32),
    mesh=plsc.VectorSubcoreMesh(core_axis_name="c", subcore_axis_name="s",
                                num_cores=NC, num_subcores=NS),
    scratch_types=(
        [
            pltpu.VMEM((E_PER_W,), jnp.int32),
            pltpu.VMEM((E_PER_W,), jnp.int32),
        ]
        + [pltpu.VMEM((CHUNK, D), jnp.float32)] * 8
        + [pltpu.SemaphoreType.DMA] * 8
    ),
  )


@jax.jit
def _run(nodes, edges, senders, receivers, W_edge, b_edge, W_node, b_node):
    w1 = W_edge[:D_EDGE]
    w_es = W_edge[D_EDGE:D_EDGE + D]
    w_er = W_edge[D_EDGE + D:]
    wn1 = W_node[:D]
    wn2 = W_node[D:]
    be = b_edge.reshape(1, D)
    bn = b_node.reshape(1, D)

    nb = 1000
    ps, pr = pl.pallas_call(
        _proj_body,
        grid=(N_NODES // nb,),
        in_specs=[
            pl.BlockSpec((nb, D), lambda i: (i, 0)),
            pl.BlockSpec((D, D), lambda i: (0, 0)),
            pl.BlockSpec((D, D), lambda i: (0, 0)),
        ],
        out_specs=[
            pl.BlockSpec((nb, D), lambda i: (i, 0)),
            pl.BlockSpec((nb, D), lambda i: (i, 0)),
        ],
        out_shape=[
            jax.ShapeDtypeStruct((N_NODES, D), jnp.float32),
            jax.ShapeDtypeStruct((N_NODES, D), jnp.float32),
        ],
    )(nodes, w_es, w_er)

    eb = 3200
    e_base = pl.pallas_call(
        _ebase_body,
        grid=(N_EDGES // eb,),
        in_specs=[
            pl.BlockSpec((eb, D_EDGE), lambda i: (i, 0)),
            pl.BlockSpec((D_EDGE, D), lambda i: (0, 0)),
            pl.BlockSpec((1, D), lambda i: (0, 0)),
        ],
        out_specs=pl.BlockSpec((eb, D), lambda i: (i, 0)),
        out_shape=jax.ShapeDtypeStruct((N_EDGES, D), jnp.float32),
    )(edges, w1, be)

    new_edges = _get_sc_edges()(ps, pr, e_base, senders, receivers)

    r3 = receivers.reshape(AGG_NB, 1, AGG_BE)
    agg = pl.pallas_call(
        _agg_body,
        grid=(AGG_NB,),
        in_specs=[
            pl.BlockSpec((1, 1, AGG_BE), lambda i: (i, 0, 0)),
            pl.BlockSpec((AGG_BE, D), lambda i: (i, 0)),
        ],
        out_specs=pl.BlockSpec((N_NODES, D), lambda i: (0, 0)),
        out_shape=jax.ShapeDtypeStruct((N_NODES, D), jnp.float32),
    )(r3, new_edges)

    new_nodes = pl.pallas_call(
        _node_body,
        grid=(N_NODES // nb,),
        in_specs=[
            pl.BlockSpec((nb, D), lambda i: (i, 0)),
            pl.BlockSpec((nb, D), lambda i: (i, 0)),
            pl.BlockSpec((D, D), lambda i: (0, 0)),
            pl.BlockSpec((D, D), lambda i: (0, 0)),
            pl.BlockSpec((1, D), lambda i: (0, 0)),
        ],
        out_specs=pl.BlockSpec((nb, D), lambda i: (i, 0)),
        out_shape=jax.ShapeDtypeStruct((N_NODES, D), jnp.float32),
    )(nodes, agg, wn1, wn2, bn)

    return new_nodes, new_edges


def kernel(nodes, edges, senders, receivers, W_edge, b_edge, W_node, b_node):
    return _run(nodes, edges, senders, receivers,
                W_edge, b_edge, W_node, b_node)
